# baseline (speedup 1.0000x reference)
import jax
import jax.numpy as jnp
from jax import lax
from jax.experimental import pallas as pl
from jax.experimental.pallas import tpu as pltpu
from jax.experimental.pallas import tpu_sc as plsc

NC, NS = 2, 16
NW = NC * NS
B = 4096 * 200
D_I32 = 32
UNIT = 128
NUNIT = 1000000 // 4
B_PER_W = B // NW
CHUNK = 320
NCHUNK = B_PER_W // CHUNK
NBUF = 2
NGROUP = NCHUNK // NBUF
G16 = CHUNK // 16


def _gather_body(idx_hbm, tab_hbm, out_hbm,
                 idx0, idx1, uidx0, uidx1, rows0, rows1, st0, st1,
                 is0, is1, gs0, gs1, os0, os1):
    wid = lax.axis_index("s") * NC + lax.axis_index("c")
    base = wid * B_PER_W
    ubase = wid * (B_PER_W // 4)
    idx_v = (idx0, idx1)
    uidx_v = (uidx0, uidx1)
    rows_v = (rows0, rows1)
    st_v = (st0, st1)
    isem = (is0, is1)
    gsem = (gs0, gs1)
    osem = (os0, os1)
    iota = lax.iota(jnp.int32, 16)

    def fire_idx(off, b):
        pltpu.async_copy(idx_hbm.at[pl.ds(off, CHUNK)], idx_v[b], isem[b])

    def wait_idx(off, b):
        pltpu.make_async_copy(idx_hbm.at[pl.ds(off, CHUNK)], idx_v[b],
                              isem[b]).wait()

    def wait_out(uoff, b):
        pltpu.make_async_copy(st_v[b], out_hbm.at[pl.ds(uoff, CHUNK // 4)],
                              osem[b]).wait()

    def fire_out(uoff, b):
        pltpu.async_copy(st_v[b], out_hbm.at[pl.ds(uoff, CHUNK // 4)],
                         osem[b])

    def gather_extract(b):
        def shift_step(g, carry):
            iv = idx_v[b][pl.ds(g * 16, 16)]
            uidx_v[b][pl.ds(g * 16, 16)] = lax.shift_right_logical(iv, 2)
            return carry

        lax.fori_loop(0, G16, shift_step, 0)
        pltpu.async_copy(tab_hbm.at[uidx_v[b]], rows_v[b], gsem[b]).wait()

        def extract_step(g, carry):
            j_vec = g * 16 + iota
            iv = idx_v[b][pl.ds(g * 16, 16)]
            sub = lax.shift_left(lax.bitwise_and(iv, 3), 5)
            for k in range(D_I32):
                val = plsc.load_gather(rows_v[b], [j_vec, sub + k])
                o = j_vec * D_I32 + k
                plsc.store_scatter(
                    st_v[b],
                    [lax.shift_right_logical(o, 7),
                     lax.bitwise_and(o, 127)], val)
            return carry

        lax.fori_loop(0, G16, extract_step, 0)

    for b in range(NBUF):
        fire_idx(base + b * CHUNK, b)
    for b in range(NBUF):
        off = base + b * CHUNK
        wait_idx(off, b)
        gather_extract(b)
        fire_idx(off + NBUF * CHUNK, b)
        fire_out(ubase + b * (CHUNK // 4), b)

    def group(g, carry):
        i0 = g * NBUF
        for b in range(NBUF):
            i = i0 + b
            off = base + i * CHUNK
            uoff = ubase + i * (CHUNK // 4)
            wait_out(uoff, b)
            wait_idx(off, b)
            gather_extract(b)
            fire_idx(off + NBUF * CHUNK, b)
            fire_out(uoff, b)
        return carry

    lax.fori_loop(1, NGROUP - 1, group, 0)

    for b in range(NBUF):
        i = (NGROUP - 1) * NBUF + b
        off = base + i * CHUNK
        wait_out(ubase + i * (CHUNK // 4), b)
        wait_idx(off, b)
        gather_extract(b)
        fire_out(ubase + i * (CHUNK // 4), b)

    for b in range(NBUF):
        wait_out(ubase + b * (CHUNK // 4), b)


def kernel(input_ids, weight):
    idx = input_ids.reshape(-1).astype(jnp.int32)
    tab = lax.bitcast_convert_type(
        weight.reshape(weight.shape[0], D_I32, 2),
        jnp.int32).reshape(NUNIT, UNIT)

    run = pl.kernel(
        _gather_body,
        out_type=jax.ShapeDtypeStruct((B // 4, UNIT), jnp.int32),
        mesh=plsc.VectorSubcoreMesh(
            core_axis_name="c", subcore_axis_name="s",
            num_cores=NC, num_subcores=NS),
        scratch_types=[
            pltpu.VMEM((CHUNK,), jnp.int32),
            pltpu.VMEM((CHUNK,), jnp.int32),
            pltpu.VMEM((CHUNK,), jnp.int32),
            pltpu.VMEM((CHUNK,), jnp.int32),
            pltpu.VMEM((CHUNK, UNIT), jnp.int32),
            pltpu.VMEM((CHUNK, UNIT), jnp.int32),
            pltpu.VMEM((CHUNK // 4, UNIT), jnp.int32),
            pltpu.VMEM((CHUNK // 4, UNIT), jnp.int32),
            pltpu.SemaphoreType.DMA,
            pltpu.SemaphoreType.DMA,
            pltpu.SemaphoreType.DMA,
            pltpu.SemaphoreType.DMA,
            pltpu.SemaphoreType.DMA,
            pltpu.SemaphoreType.DMA,
        ],
        compiler_params=pltpu.CompilerParams(
            use_tc_tiling_on_sc=True, needs_layout_passes=False),
    )
    out = run(idx, tab)
    hidden = lax.bitcast_convert_type(out, jnp.bfloat16)
    return hidden.reshape(input_ids.shape[0], input_ids.shape[1], 64)


# speedup vs baseline: 8.1903x; 8.1903x over previous
import jax
import jax.numpy as jnp
from jax import lax
from jax.experimental import pallas as pl
from jax.experimental.pallas import tpu as pltpu
from jax.experimental.pallas import tpu_sc as plsc

NC, NS = 2, 16
NW = NC * NS
B = 4096 * 200
D_I32 = 32
B_PER_W = B // NW
CHUNK = 1600
NCHUNK = B_PER_W // CHUNK
NBUF = 2
NGROUP = NCHUNK // NBUF


def _gather_body(idx_hbm, tab_hbm, out_hbm,
                 idx0, idx1, rows0, rows1,
                 is0, is1, gs0, gs1, os0, os1):
    wid = lax.axis_index("s") * NC + lax.axis_index("c")
    base = wid * B_PER_W
    idx_v = (idx0, idx1)
    rows_v = (rows0, rows1)
    isem = (is0, is1)
    gsem = (gs0, gs1)
    osem = (os0, os1)

    def fire_idx(off, b):
        pltpu.async_copy(idx_hbm.at[pl.ds(off, CHUNK)], idx_v[b], isem[b])

    def wait_idx(off, b):
        pltpu.make_async_copy(idx_hbm.at[pl.ds(off, CHUNK)], idx_v[b],
                              isem[b]).wait()

    def wait_out(off, b):
        pltpu.make_async_copy(rows_v[b], out_hbm.at[pl.ds(off, CHUNK)],
                              osem[b]).wait()

    def gather(b):
        pltpu.async_copy(tab_hbm.at[idx_v[b]], rows_v[b], gsem[b]).wait()

    def fire_out(off, b):
        pltpu.async_copy(rows_v[b], out_hbm.at[pl.ds(off, CHUNK)], osem[b])

    for b in range(NBUF):
        fire_idx(base + b * CHUNK, b)
    for b in range(NBUF):
        off = base + b * CHUNK
        wait_idx(off, b)
        gather(b)
        fire_idx(off + NBUF * CHUNK, b)
        fire_out(off, b)

    def group(g, carry):
        i0 = g * NBUF
        for b in range(NBUF):
            off = base + (i0 + b) * CHUNK
            wait_out(off, b)
            wait_idx(off, b)
            gather(b)
            fire_idx(off + NBUF * CHUNK, b)
            fire_out(off, b)
        return carry

    lax.fori_loop(1, NGROUP - 1, group, 0)

    for b in range(NBUF):
        off = base + ((NGROUP - 1) * NBUF + b) * CHUNK
        wait_out(off, b)
        wait_idx(off, b)
        gather(b)
        fire_out(off, b)

    for b in range(NBUF):
        wait_out(base + b * CHUNK, b)


def kernel(input_ids, weight):
    idx = input_ids.reshape(-1).astype(jnp.int32)
    tab = lax.bitcast_convert_type(
        weight.reshape(weight.shape[0], D_I32, 2), jnp.int32)

    run = pl.kernel(
        _gather_body,
        out_type=jax.ShapeDtypeStruct((B, D_I32), jnp.int32),
        mesh=plsc.VectorSubcoreMesh(
            core_axis_name="c", subcore_axis_name="s",
            num_cores=NC, num_subcores=NS),
        scratch_types=[
            pltpu.VMEM((CHUNK,), jnp.int32),
            pltpu.VMEM((CHUNK,), jnp.int32),
            pltpu.VMEM((CHUNK, D_I32), jnp.int32),
            pltpu.VMEM((CHUNK, D_I32), jnp.int32),
            pltpu.SemaphoreType.DMA,
            pltpu.SemaphoreType.DMA,
            pltpu.SemaphoreType.DMA,
            pltpu.SemaphoreType.DMA,
            pltpu.SemaphoreType.DMA,
            pltpu.SemaphoreType.DMA,
        ],
        compiler_params=pltpu.CompilerParams(use_tc_tiling_on_sc=False),
    )
    out = run(idx, tab)
    hidden = lax.bitcast_convert_type(out, jnp.bfloat16)
    return hidden.reshape(input_ids.shape[0], input_ids.shape[1], 64)
